# Initial kernel scaffold; baseline (speedup 1.0000x reference)
import jax
import jax.numpy as jnp
from jax.experimental import pallas as pl


def kernel(pred_ab, pred_ang, pred_hm, target_ab, target_ang, target_hm, ind, reg_mask):
    raise NotImplementedError("write your pallas kernel here")


# speedup vs baseline: 1.3466x; 1.3466x over previous
import functools
import math

import jax
import jax.numpy as jnp
from jax.experimental import pallas as pl

_F32 = jnp.float32
_DEG2RAD = math.pi / 180.0


def _gwd_kernel(pred_ab_ref, pred_ang_ref, misc_ref, out_ref):
    b = pl.program_id(0)

    featA = pred_ab_ref[0, 0]
    featB = pred_ab_ref[0, 1]
    featG = pred_ang_ref[0, 0]
    m = misc_ref[0]

    ys_p = m[:, 0:1]
    xs_p = m[:, 1:2]
    ys_t = m[:, 2:3]
    xs_t = m[:, 3:4]
    ta = m[:, 4:5]
    tb = m[:, 5:6]
    tang = m[:, 6:7]
    mask = m[:, 7:8]
    row = m[:, 8:9]
    col = m[:, 9:10]

    li = jax.lax.broadcasted_iota(jnp.int32, (1, 128), 1).astype(_F32)
    ohr = (li == row).astype(_F32)
    ohc = (li == col).astype(_F32)

    featcat = jnp.concatenate([featA, featB, featG], axis=1)
    rows_sel = jnp.dot(ohr, featcat, preferred_element_type=_F32)
    a_g = jnp.sum(rows_sel[:, 0:128] * ohc, axis=1, keepdims=True)
    b_g = jnp.sum(rows_sel[:, 128:256] * ohc, axis=1, keepdims=True)
    g_g = jnp.sum(rows_sel[:, 256:384] * ohc, axis=1, keepdims=True)

    yp = ys_p * mask
    xp = xs_p * mask
    wp = a_g * 2.0 * mask
    hp = b_g * 2.0 * mask
    angp = (g_g - 90.0) * mask

    yt = ys_t * mask
    xt = xs_t * mask
    wt = ta * 2.0 * mask
    ht = tb * 2.0 * mask
    angt = (tang - 90.0) * mask

    xy_dist = jnp.square(yp - yt) + jnp.square(xp - xt)

    wp_ = jnp.clip(wp, 1e-07, 10000000.0)
    hp_ = jnp.clip(hp, 1e-07, 10000000.0)
    wt_ = jnp.clip(wt, 1e-07, 10000000.0)
    ht_ = jnp.clip(ht, 1e-07, 10000000.0)

    rp = angp * _DEG2RAD
    rt = angt * _DEG2RAD
    cp = jnp.cos(rp)
    sp = jnp.sin(rp)
    ct = jnp.cos(rt)
    st = jnp.sin(rt)

    dap = 0.5 * wp_
    dbp = 0.5 * hp_
    dat = 0.5 * wt_
    dbt = 0.5 * ht_
    a2p = dap * dap
    b2p = dbp * dbp
    a2t = dat * dat
    b2t = dbt * dbt

    Sp11 = cp * cp * a2p + sp * sp * b2p
    Sp22 = sp * sp * a2p + cp * cp * b2p
    Sp12 = cp * sp * (a2p - b2p)
    St11 = ct * ct * a2t + st * st * b2t
    St22 = st * st * a2t + ct * ct * b2t
    St12 = ct * st * (a2t - b2t)

    tr = Sp11 * St11 + 2.0 * Sp12 * St12 + Sp22 * St22
    det_sqrt = (dap * dbp) * (dat * dbt)
    whr = (a2p + b2p) + (a2t + b2t)
    whr = whr - 2.0 * jnp.sqrt(jnp.clip(tr + 2.0 * det_sqrt, 0.0, None))

    distance = jnp.clip(xy_dist + whr, 0.0, None)
    distance = jnp.log1p(distance)
    lossv = 1.0 - 1.0 / (1.0 + distance)

    lsum = jnp.sum(lossv)
    msum = jnp.sum(mask)

    lane = jax.lax.broadcasted_iota(jnp.int32, (1, 128), 1)

    @pl.when(b == 0)
    def _():
        out_ref[...] = jnp.zeros((1, 128), _F32)

    out_ref[...] += jnp.where(lane == 0, lsum, 0.0) + jnp.where(lane == 1, msum, 0.0)


def kernel(pred_ab, pred_ang, pred_hm, target_ab, target_ang, target_hm, ind, reg_mask):
    B, C, H, W = pred_ab.shape
    K = ind.shape[1]
    HW = H * W

    flat_p = pred_hm.reshape(B, HW)
    _, ind_p = jax.lax.top_k(flat_p, K)
    ys_p = (ind_p // W).astype(_F32)
    xs_p = (ind_p % W).astype(_F32)

    flat_t = target_hm.reshape(B, HW)
    _, ind_t = jax.lax.top_k(flat_t, K)
    ys_t = (ind_t // W).astype(_F32)
    xs_t = (ind_t % W).astype(_F32)

    mask = reg_mask.astype(_F32)
    ind32 = ind.astype(jnp.int32)
    row_f = (ind32 // W).astype(_F32)
    col_f = (ind32 % W).astype(_F32)

    fields = [
        ys_p, xs_p, ys_t, xs_t,
        target_ab[:, :, 0], target_ab[:, :, 1], target_ang[:, :, 0],
        mask, row_f, col_f,
    ]
    misc = jnp.stack(fields, axis=2)
    misc = jnp.pad(misc, ((0, 0), (0, 128 - K), (0, 16 - misc.shape[2])))

    out = pl.pallas_call(
        _gwd_kernel,
        grid=(B,),
        in_specs=[
            pl.BlockSpec((1, C, H, W), lambda b: (b, 0, 0, 0)),
            pl.BlockSpec((1, 1, H, W), lambda b: (b, 0, 0, 0)),
            pl.BlockSpec((1, 128, 16), lambda b: (b, 0, 0)),
        ],
        out_specs=pl.BlockSpec((1, 128), lambda b: (0, 0)),
        out_shape=jax.ShapeDtypeStruct((1, 128), _F32),
    )(pred_ab, pred_ang, misc)

    return out[0, 0] / (out[0, 1] + 1e-08)
